# baseline (speedup 1.0000x reference)
import functools
import math

import jax
import jax.numpy as jnp
from jax import lax
from jax.experimental import pallas as pl
from jax.experimental.pallas import tpu as pltpu
from jax.experimental.pallas import tpu_sc as plsc

VOCAB = 1000000
EMB = 64
SCALE = math.sqrt(EMB)

NC = 2
NS = 16
NW = NC * NS

B_TOTAL = 4096 * 200
BPW = B_TOTAL // NW
CHUNK = 128
NCHUNK = BPW // CHUNK
LANES = 16
VPR = EMB // LANES


def _emb_kernel_body(table_hbm, idx_hbm, out_hbm, idx_v, buf, sem):
    c = lax.axis_index("c")
    s = lax.axis_index("s")
    wid = s * NC + c

    pltpu.sync_copy(idx_hbm.at[wid], idx_v)

    def chunk_body(j, carry):
        pltpu.async_copy(table_hbm.at[idx_v.at[j]], buf, sem).wait()

        def row_body(r, rc):
            for cc in range(VPR):
                sl = pl.ds(cc * LANES, LANES)
                buf[r, sl] = buf[r, sl] * SCALE
            return rc

        lax.fori_loop(0, CHUNK, row_body, 0)

        pltpu.sync_copy(buf, out_hbm.at[wid, j])
        return carry

    lax.fori_loop(0, NCHUNK, chunk_body, 0)


@jax.jit
def _emb_lookup(table, idx):
    mesh = plsc.VectorSubcoreMesh(core_axis_name="c", subcore_axis_name="s")
    fn = pl.kernel(
        _emb_kernel_body,
        mesh=mesh,
        out_type=jax.ShapeDtypeStruct((NW, NCHUNK, CHUNK, EMB), jnp.float32),
        scratch_types=[
            pltpu.VMEM((NCHUNK, CHUNK), jnp.int32),
            pltpu.VMEM((CHUNK, EMB), jnp.float32),
            pltpu.SemaphoreType.DMA,
        ],
        compiler_params=pltpu.CompilerParams(use_tc_tiling_on_sc=False),
    )
    return fn(table, idx)


def kernel(tokens, table):
    idx = tokens.reshape(NW, NCHUNK, CHUNK).astype(jnp.int32)
    out = _emb_lookup(table, idx)
    return out.reshape(4096, 200, EMB)


# speedup vs baseline: 1.2081x; 1.2081x over previous
import functools
import math

import jax
import jax.numpy as jnp
from jax import lax
from jax.experimental import pallas as pl
from jax.experimental.pallas import tpu as pltpu
from jax.experimental.pallas import tpu_sc as plsc

VOCAB = 1000000
EMB = 64
SCALE = math.sqrt(EMB)

NC = 2
NS = 16
NW = NC * NS

B_TOTAL = 4096 * 200
BPW = B_TOTAL // NW
CHUNK = 128
NCHUNK = BPW // CHUNK
LANES = 16
VPR = EMB // LANES


NBUF = 4
NGROUP = NCHUNK // NBUF


def _emb_kernel_body(table_hbm, idx_hbm, out_hbm, idx_v, in_bufs, out_bufs,
                     gsem, ssem):
    c = lax.axis_index("c")
    s = lax.axis_index("s")
    wid = s * NC + c

    pltpu.sync_copy(idx_hbm.at[wid], idx_v)

    for b in range(NBUF):
        pltpu.async_copy(table_hbm.at[idx_v.at[b]], in_bufs.at[b], gsem.at[b])

    def group_body(g, carry):
        for b in range(NBUF):
            j = g * NBUF + b

            @pl.when(g > 0)
            def _():
                pltpu.make_async_copy(out_bufs.at[b], out_hbm.at[wid, j],
                                      ssem.at[b]).wait()

            pltpu.make_async_copy(table_hbm.at[idx_v.at[j]], in_bufs.at[b],
                                  gsem.at[b]).wait()

            def row_body(r, rc):
                for cc in range(VPR):
                    sl = pl.ds(cc * LANES, LANES)
                    out_bufs[b, r, sl] = in_bufs[b, r, sl] * SCALE
                return rc

            lax.fori_loop(0, CHUNK, row_body, 0)

            pltpu.async_copy(out_bufs.at[b], out_hbm.at[wid, j], ssem.at[b])

            @pl.when(j + NBUF < NCHUNK)
            def _():
                pltpu.async_copy(table_hbm.at[idx_v.at[j + NBUF]],
                                 in_bufs.at[b], gsem.at[b])

        return carry

    lax.fori_loop(0, NGROUP, group_body, 0)

    for b in range(NBUF):
        pltpu.make_async_copy(out_bufs.at[b], out_hbm.at[wid, NCHUNK - NBUF + b],
                              ssem.at[b]).wait()


@jax.jit
def _emb_lookup(table, idx):
    mesh = plsc.VectorSubcoreMesh(core_axis_name="c", subcore_axis_name="s")
    fn = pl.kernel(
        _emb_kernel_body,
        mesh=mesh,
        out_type=jax.ShapeDtypeStruct((NW, NCHUNK, CHUNK, EMB), jnp.float32),
        scratch_types=[
            pltpu.VMEM((NCHUNK, CHUNK), jnp.int32),
            pltpu.VMEM((NBUF, CHUNK, EMB), jnp.float32),
            pltpu.VMEM((NBUF, CHUNK, EMB), jnp.float32),
            pltpu.SemaphoreType.DMA((NBUF,)),
            pltpu.SemaphoreType.DMA((NBUF,)),
        ],
        compiler_params=pltpu.CompilerParams(use_tc_tiling_on_sc=False),
    )
    return fn(table, idx)


def kernel(tokens, table):
    idx = tokens.reshape(NW, NCHUNK, CHUNK).astype(jnp.int32)
    out = _emb_lookup(table, idx)
    return out.reshape(4096, 200, EMB)


# speedup vs baseline: 1.2101x; 1.0017x over previous
import functools
import math

import jax
import jax.numpy as jnp
from jax import lax
from jax.experimental import pallas as pl
from jax.experimental.pallas import tpu as pltpu
from jax.experimental.pallas import tpu_sc as plsc

VOCAB = 1000000
EMB = 64
SCALE = math.sqrt(EMB)

NC = 2
NS = 16
NW = NC * NS

B_TOTAL = 4096 * 200
BPW = B_TOTAL // NW
CHUNK = 128
NCHUNK = BPW // CHUNK
LANES = 16
VPR = EMB // LANES


NBUF = 4
NGROUP = NCHUNK // NBUF


def _emb_kernel_body(table_hbm, idx_hbm, out_hbm, idx_v, in_bufs, out_bufs,
                     gsem, ssem):
    c = lax.axis_index("c")
    s = lax.axis_index("s")
    wid = s * NC + c

    pltpu.sync_copy(idx_hbm.at[wid], idx_v)

    for b in range(NBUF):
        pltpu.async_copy(table_hbm.at[idx_v.at[b]], in_bufs.at[b], gsem.at[b])

    def group_body(g, carry):
        for b in range(NBUF):
            j = g * NBUF + b

            @pl.when(g > 0)
            def _():
                pltpu.make_async_copy(out_bufs.at[b], out_hbm.at[wid, j],
                                      ssem.at[b]).wait()

            pltpu.make_async_copy(table_hbm.at[idx_v.at[j]], in_bufs.at[b],
                                  gsem.at[b]).wait()

            @plsc.parallel_loop(0, CHUNK, step=1, unroll=8)
            def _(r):
                for cc in range(VPR):
                    sl = pl.ds(cc * LANES, LANES)
                    out_bufs[b, r, sl] = in_bufs[b, r, sl] * SCALE

            pltpu.async_copy(out_bufs.at[b], out_hbm.at[wid, j], ssem.at[b])

            @pl.when(j + NBUF < NCHUNK)
            def _():
                pltpu.async_copy(table_hbm.at[idx_v.at[j + NBUF]],
                                 in_bufs.at[b], gsem.at[b])

        return carry

    lax.fori_loop(0, NGROUP, group_body, 0)

    for b in range(NBUF):
        pltpu.make_async_copy(out_bufs.at[b], out_hbm.at[wid, NCHUNK - NBUF + b],
                              ssem.at[b]).wait()


@jax.jit
def _emb_lookup(table, idx):
    mesh = plsc.VectorSubcoreMesh(core_axis_name="c", subcore_axis_name="s")
    fn = pl.kernel(
        _emb_kernel_body,
        mesh=mesh,
        out_type=jax.ShapeDtypeStruct((NW, NCHUNK, CHUNK, EMB), jnp.float32),
        scratch_types=[
            pltpu.VMEM((NCHUNK, CHUNK), jnp.int32),
            pltpu.VMEM((NBUF, CHUNK, EMB), jnp.float32),
            pltpu.VMEM((NBUF, CHUNK, EMB), jnp.float32),
            pltpu.SemaphoreType.DMA((NBUF,)),
            pltpu.SemaphoreType.DMA((NBUF,)),
        ],
        compiler_params=pltpu.CompilerParams(use_tc_tiling_on_sc=False),
    )
    return fn(table, idx)


def kernel(tokens, table):
    idx = tokens.reshape(NW, NCHUNK, CHUNK).astype(jnp.int32)
    out = _emb_lookup(table, idx)
    return out.reshape(4096, 200, EMB)
